# baseline (speedup 1.0000x reference)
import functools

import jax
import jax.numpy as jnp
from jax import lax
from jax.experimental import pallas as pl
from jax.experimental.pallas import tpu as pltpu
from jax.experimental.pallas import tpu_sc as plsc

N = 10000
E2 = 320000
L = 128
EB = 512
EPAD = 401408
NBLK = EPAD // EB
W = 72
NPAD = 10112
NB = 128
NNBLK = NPAD // NB
STEPS = 15


def _ln_rows(y):
    mu = jnp.mean(y, axis=-1, keepdims=True)
    var = jnp.mean((y - mu) * (y - mu), axis=-1, keepdims=True)
    return (y - mu) * lax.rsqrt(var + 1e-5)


NBUF = 4


def _sc_gather(table, idx):
    B = idx.shape[0]
    D = table.shape[1]
    info = plsc.get_sparse_core_info()
    nc, ns = info.num_cores, info.num_subcores
    nw = nc * ns
    bpw = B // nw
    ch = 128
    nch = bpw // ch
    idx3 = idx.reshape(nw, nch, ch)
    mesh = plsc.VectorSubcoreMesh(core_axis_name="c", subcore_axis_name="s")

    @functools.partial(
        pl.kernel,
        mesh=mesh,
        out_type=jax.ShapeDtypeStruct((B, D), jnp.float32),
        scratch_types=[
            pltpu.VMEM_SHARED((table.shape[0], D), jnp.float32),
            pltpu.VMEM((ch,), jnp.int32),
            pltpu.VMEM((ch,), jnp.int32),
            pltpu.VMEM((ch, D), jnp.float32),
            pltpu.VMEM((ch, D), jnp.float32),
            pltpu.SemaphoreType.DMA,
            pltpu.SemaphoreType.DMA,
            pltpu.SemaphoreType.DMA,
            pltpu.SemaphoreType.DMA,
            pltpu.SemaphoreType.DMA,
            pltpu.SemaphoreType.DMA,
        ],
    )
    def k(table_hbm, idx_hbm, out_hbm, table_sh, i0, i1, r0, r1,
          gi0, gi1, g0, g1, o0, o1):
        wid = lax.axis_index("s") * nc + lax.axis_index("c")
        base = wid * bpw
        idxs = (i0, i1)
        rows = (r0, r1)
        gi = (gi0, gi1)
        g = (g0, g1)
        o = (o0, o1)

        @pl.when(lax.axis_index("s") == 0)
        def _():
            pltpu.sync_copy(table_hbm, table_sh)
        plsc.subcore_barrier()

        def start_idx(j, b):
            pltpu.async_copy(idx_hbm.at[wid, j], idxs[b], gi[b])

        def wait_idx(b):
            pltpu.make_async_copy(idx_hbm.at[wid, 0], idxs[b], gi[b]).wait()

        def start_gather(b):
            pltpu.async_copy(table_sh.at[idxs[b]], rows[b], g[b])

        def wait_gather(b):
            pltpu.make_async_copy(table_hbm.at[i0], rows[b], g[b]).wait()

        def start_out(j, b):
            pltpu.async_copy(rows[b], out_hbm.at[pl.ds(base + j * ch, ch)],
                             o[b])

        def wait_out(b):
            pltpu.make_async_copy(table_hbm.at[i0], rows[b], o[b]).wait()

        start_idx(0, 0)
        start_idx(1, 1)

        def chunk(j, b):
            wait_idx(b)

            @pl.when(j >= 2)
            def _():
                wait_out(b)

            start_gather(b)
            wait_gather(b)
            start_out(j, b)

            @pl.when(j + 2 < nch)
            def _():
                start_idx(j + 2, b)

        def body(i, _):
            for b in (0, 1):
                chunk(i * 2 + b, b)
            return 0

        lax.fori_loop(0, nch // 2, body, 0, unroll=False)
        pltpu.make_async_copy(table_hbm.at[i0], rows[0], o[0]).wait()
        pltpu.make_async_copy(table_hbm.at[i0], rows[1], o[1]).wait()

    return k(table, idx3)


def _backbone_body(x_ref, w_ref, b_ref, fcw_ref, fcb_ref, pw_ref, pb_ref,
                   iv_ref, acc_ref):
    c = pl.program_id(0)

    @pl.when(c == 0)
    def _():
        acc_ref[...] = jnp.zeros_like(acc_ref)

    h = jnp.maximum(jnp.dot(x_ref[...], w_ref[...]) + b_ref[...], 0.0)
    acc_ref[...] += jnp.dot(jnp.ones((8, x_ref.shape[0]), jnp.float32), h)

    @pl.when(c == pl.num_programs(0) - 1)
    def _():
        m = acc_ref[...] * (1.0 / 3136.0)
        hh = jnp.maximum(jnp.dot(m, fcw_ref[...]) + fcb_ref[...], 0.0)
        iv_ref[...] = jnp.dot(hh, pw_ref[...]) + pb_ref[...]


def _node_enc_body(mp_ref, w1_ref, b1_ref, w2_ref, b2_ref, iv_ref,
                   ws_ref, wr_ref, x_ref, xs_ref, xr_ref):
    t = jnp.maximum(jnp.dot(mp_ref[...], w1_ref[...]) + b1_ref[...], 0.0)
    y = jnp.dot(t, w2_ref[...]) + b2_ref[...]
    x = _ln_rows(y) + iv_ref[0:1, :]
    x_ref[...] = x
    xs_ref[...] = jnp.dot(x, ws_ref[...])
    xr_ref[...] = jnp.dot(x, wr_ref[...])


def _edge_enc_body(wstart_ref, ms_ref, r_ref, mp_ref, w1_ref, b1_ref,
                   w2_ref, b2_ref, e0_ref):
    c = pl.program_id(0)
    ws = wstart_ref[c]
    r_loc = r_ref[...] - ws
    oh = (r_loc == lax.broadcasted_iota(jnp.int32, (EB, W), 1))
    mr = jnp.dot(oh.astype(jnp.float32), mp_ref[pl.ds(ws, W), :])
    d = ms_ref[:, 0:16] - mr
    nrm = jnp.sqrt(jnp.sum(d * d, axis=-1, keepdims=True))
    col3 = (lax.broadcasted_iota(jnp.int32, (EB, 16), 1) == 3)
    ef = d + jnp.where(col3, nrm, 0.0)
    t = jnp.maximum(jnp.dot(ef, w1_ref[...]) + b1_ref[...], 0.0)
    y = jnp.dot(t, w2_ref[...]) + b2_ref[...]
    e0_ref[...] = _ln_rows(y)


def _edge_step_body(wstart_ref, e_ref, gs_ref, r_ref, m_ref, xr_ref,
                    we_ref, w2_ref, b1_ref, b2_ref, aw_ref,
                    enew_ref, num_ref, den_ref):
    c = pl.program_id(0)

    @pl.when(c == 0)
    def _():
        num_ref[...] = jnp.zeros_like(num_ref)
        den_ref[...] = jnp.zeros_like(den_ref)

    ws = wstart_ref[c]
    e = e_ref[...]
    r_loc = r_ref[...] - ws
    mask = m_ref[...]
    oh = (r_loc == lax.broadcasted_iota(jnp.int32, (EB, W), 1))
    oh = oh.astype(jnp.float32)
    xr_win = xr_ref[pl.ds(ws, W), :]
    hp = lax.Precision.HIGHEST
    gr = jnp.dot(oh, xr_win, precision=hp)
    t = jnp.maximum(jnp.dot(e, we_ref[...], precision=hp) + gs_ref[...]
                    + gr + b1_ref[...], 0.0)
    y = jnp.dot(t, w2_ref[...], precision=hp) + b2_ref[...]
    e_new = _ln_rows(y) + e
    enew_ref[...] = e_new
    l = jnp.sum(e_new * aw_ref[...], axis=-1, keepdims=True)
    l = jnp.where(l >= 0, l, 0.2 * l)
    u = jnp.where(mask > 0.5, jnp.exp(l), 0.0)
    w = e_new * u
    dn = (((0,), (0,)), ((), ()))
    pnum = lax.dot_general(oh, w, dn, precision=hp)
    pden = lax.dot_general(oh, jnp.broadcast_to(u, (EB, 8)), dn,
                           precision=hp)
    num_ref[pl.ds(ws, W), :] += pnum
    den_ref[pl.ds(ws, W), :] += pden


def _node_step_body(num_ref, den_ref, x_ref, wx_ref, wa_ref, b1_ref,
                    w2_ref, b2_ref, ws_ref, wr_ref,
                    xn_ref, xs_ref, xr_ref):
    hp = lax.Precision.HIGHEST
    agg = num_ref[...] / jnp.maximum(den_ref[:, 0:1], 1e-20)
    x = x_ref[...]
    t = jnp.maximum(jnp.dot(x, wx_ref[...], precision=hp)
                    + jnp.dot(agg, wa_ref[...], precision=hp)
                    + b1_ref[...], 0.0)
    y = jnp.dot(t, w2_ref[...], precision=hp) + b2_ref[...]
    x_new = _ln_rows(y) + x
    xn_ref[...] = x_new
    xs_ref[...] = jnp.dot(x_new, ws_ref[...], precision=hp)
    xr_ref[...] = jnp.dot(x_new, wr_ref[...], precision=hp)


def _dec_body(x_ref, w1_ref, b1_ref, w2_ref, b2_ref, o_ref):
    t = jnp.maximum(jnp.dot(x_ref[...], w1_ref[...]) + b1_ref[...], 0.0)
    o_ref[...] = jnp.dot(t, w2_ref[...]) + b2_ref[...]


def _const2(shape):
    return pl.BlockSpec(shape, lambda *_: (0, 0))


def _rowblk(bs, nl):
    return pl.BlockSpec((bs, nl), lambda c, *_: (c, 0))


def kernel(batch, mesh_pos, edge_idx, params):
    p = params
    f32 = jnp.float32

    senders = jnp.concatenate([edge_idx[:, 0], edge_idx[:, 1]], axis=0)
    receivers = jnp.concatenate([edge_idx[:, 1], edge_idx[:, 0]], axis=0)
    order = jnp.argsort(receivers)
    r_s = receivers[order].astype(jnp.int32)
    s_s = senders[order].astype(jnp.int32)
    deg = jnp.zeros((N,), jnp.int32).at[receivers].add(1)
    start = jnp.cumsum(deg) - deg
    pd = jnp.maximum(8, ((deg + 7) // 8) * 8)
    cum = jnp.cumsum(pd)
    slot0 = cum - pd
    pos = jnp.arange(E2, dtype=jnp.int32) + (slot0 - start)[r_s]
    r_pad = jax.lax.cummax(
        jnp.zeros((EPAD,), jnp.int32).at[slot0].set(
            jnp.arange(N, dtype=jnp.int32)))
    sm = jnp.zeros((EPAD,), jnp.int32).at[pos].set(s_s + 1)
    s_pad = jnp.maximum(sm - 1, 0)
    maskf = (sm > 0).astype(f32)
    wstart = ((r_pad[::EB] // 8) * 8).astype(jnp.int32)
    r_col = r_pad.reshape(EPAD, 1)
    m_col = maskf.reshape(EPAD, 1)

    we = p['mp_e_w1'][0:L]
    wsnd = p['mp_e_w1'][L:2 * L]
    wrcv = p['mp_e_w1'][2 * L:3 * L]
    wnx = p['mp_n_w1'][0:L]
    wna = p['mp_n_w1'][L:2 * L]
    b1e = p['mp_e_b1'].reshape(1, L)
    b2e = p['mp_e_b2'].reshape(1, L)
    b1n = p['mp_n_b1'].reshape(1, L)
    b2n = p['mp_n_b2'].reshape(1, L)
    aw = p['att_w'].reshape(1, L)
    enw1 = jnp.zeros((16, L), f32).at[0:3].set(p['enc_n_w1'])
    enb1 = p['enc_n_b1'].reshape(1, L)
    enw2 = p['enc_n_w2']
    enb2 = p['enc_n_b2'].reshape(1, L)
    eew1 = jnp.zeros((16, L), f32).at[0:4].set(p['enc_e_w1'])
    eeb1 = p['enc_e_b1'].reshape(1, L)
    eew2 = p['enc_e_w2']
    eeb2 = p['enc_e_b2'].reshape(1, L)
    dw1 = p['dec_w1']
    db1 = p['dec_b1'].reshape(1, L)
    dw2 = jnp.zeros((L, L), f32).at[:, 0:3].set(p['dec_w2'])
    db2 = jnp.zeros((1, L), f32).at[0, 0:3].set(p['dec_b2'])

    xpad = jnp.pad(batch, ((0, 0), (0, 0), (1, 2), (1, 2)))
    cols = []
    for ky in range(7):
        for kx in range(7):
            cols.append(xpad[0, :, ky:ky + 221:4, kx:kx + 221:4]
                        .reshape(3, 3136))
    X = jnp.concatenate(cols, axis=0)
    Wc = jnp.transpose(p['conv_w'], (2, 3, 1, 0)).reshape(147, 64)
    Xp = jnp.zeros((3136, 160), f32).at[:, 0:147].set(X.T)
    Wp = jnp.zeros((160, 64), f32).at[0:147].set(Wc)
    cb = p['conv_b'].reshape(1, 64)
    fcw = p['fc_w']
    fcb = p['fc_b'].reshape(1, 512)
    pw = p['img_proj_w']
    pb = p['img_proj_b'].reshape(1, L)

    iv = pl.pallas_call(
        _backbone_body,
        grid=(7,),
        in_specs=[
            pl.BlockSpec((448, 160), lambda c: (c, 0)),
            _const2((160, 64)), _const2((1, 64)),
            _const2((64, 512)), _const2((1, 512)),
            _const2((512, L)), _const2((1, L)),
        ],
        out_specs=_const2((8, L)),
        out_shape=jax.ShapeDtypeStruct((8, L), f32),
        scratch_shapes=[pltpu.VMEM((8, 64), f32)],
    )(Xp, Wp, cb, fcw, fcb, pw, pb)

    mp16 = jnp.zeros((NPAD, 16), f32).at[0:N, 0:3].set(mesh_pos)
    x0, xs0, xr0 = pl.pallas_call(
        _node_enc_body,
        grid=(NNBLK,),
        in_specs=[
            _rowblk(NB, 16),
            _const2((16, L)), _const2((1, L)), _const2((L, L)), _const2((1, L)),
            _const2((8, L)), _const2((L, L)), _const2((L, L)),
        ],
        out_specs=[_rowblk(NB, L)] * 3,
        out_shape=[jax.ShapeDtypeStruct((NPAD, L), f32)] * 3,
    )(mp16, enw1, enb1, enw2, enb2, iv, wsnd, wrcv)

    mp128 = jnp.zeros((NPAD, L), f32).at[0:N, 0:3].set(mesh_pos)
    mp_s = _sc_gather(mp128, s_pad)
    e0 = pl.pallas_call(
        _edge_enc_body,
        grid_spec=pltpu.PrefetchScalarGridSpec(
            num_scalar_prefetch=1,
            grid=(NBLK,),
            in_specs=[
                _rowblk(EB, L), _rowblk(EB, 1),
                _const2((NPAD, 16)),
                _const2((16, L)), _const2((1, L)), _const2((L, L)),
                _const2((1, L)),
            ],
            out_specs=_rowblk(EB, L),
        ),
        out_shape=jax.ShapeDtypeStruct((EPAD, L), f32),
    )(wstart, mp_s, r_col, mp16, eew1, eeb1, eew2, eeb2)

    edge_call = pl.pallas_call(
        _edge_step_body,
        grid_spec=pltpu.PrefetchScalarGridSpec(
            num_scalar_prefetch=1,
            grid=(NBLK,),
            in_specs=[
                _rowblk(EB, L), _rowblk(EB, L),
                _rowblk(EB, 1), _rowblk(EB, 1),
                _const2((NPAD, L)),
                _const2((L, L)), _const2((L, L)),
                _const2((1, L)), _const2((1, L)), _const2((1, L)),
            ],
            out_specs=[
                _rowblk(EB, L),
                _const2((NPAD, L)),
                _const2((NPAD, 8)),
            ],
        ),
        out_shape=[
            jax.ShapeDtypeStruct((EPAD, L), f32),
            jax.ShapeDtypeStruct((NPAD, L), f32),
            jax.ShapeDtypeStruct((NPAD, 8), f32),
        ],
        input_output_aliases={1: 0},
    )

    node_call = pl.pallas_call(
        _node_step_body,
        grid=(NNBLK,),
        in_specs=[
            _rowblk(NB, L), _rowblk(NB, 8), _rowblk(NB, L),
            _const2((L, L)), _const2((L, L)), _const2((1, L)),
            _const2((L, L)), _const2((1, L)),
            _const2((L, L)), _const2((L, L)),
        ],
        out_specs=[_rowblk(NB, L)] * 3,
        out_shape=[jax.ShapeDtypeStruct((NPAD, L), f32)] * 3,
        input_output_aliases={2: 0},
    )

    def step(carry, _):
        x, e, xs, xr = carry
        gs = _sc_gather(xs, s_pad)
        e_new, num, den = edge_call(wstart, e, gs, r_col, m_col, xr,
                                    we, p['mp_e_w2'], b1e, b2e, aw)
        x_new, xs2, xr2 = node_call(num, den, x, wnx, wna, b1n,
                                    p['mp_n_w2'], b2n, wsnd, wrcv)
        return (x_new, e_new, xs2, xr2), None

    (x, e, xs, xr), _ = lax.scan(step, (x0, e0, xs0, xr0), None, length=STEPS)

    pred = pl.pallas_call(
        _dec_body,
        grid=(NNBLK,),
        in_specs=[
            _rowblk(NB, L),
            _const2((L, L)), _const2((1, L)), _const2((L, L)), _const2((1, L)),
        ],
        out_specs=_rowblk(NB, L),
        out_shape=jax.ShapeDtypeStruct((NPAD, L), f32),
    )(x, dw1, db1, dw2, db2)

    return pred[0:N, 0:3]


# speedup vs baseline: 1.7095x; 1.7095x over previous
import functools

import jax
import jax.numpy as jnp
from jax import lax
from jax.experimental import pallas as pl
from jax.experimental.pallas import tpu as pltpu
from jax.experimental.pallas import tpu_sc as plsc

N = 10000
E2 = 320000
L = 128
EB = 512
EPAD = 401408
NBLK = EPAD // EB
W = 72
NPAD = 10112
NB = 128
NNBLK = NPAD // NB
STEPS = 15


def _ln_rows(y):
    mu = jnp.mean(y, axis=-1, keepdims=True)
    var = jnp.mean((y - mu) * (y - mu), axis=-1, keepdims=True)
    return (y - mu) * lax.rsqrt(var + 1e-5)


NBUF = 4


def _sc_gather(table, idx):
    B = idx.shape[0]
    D = table.shape[1]
    info = plsc.get_sparse_core_info()
    nc, ns = info.num_cores, info.num_subcores
    nw = nc * ns
    bpw = B // nw
    ch = 128
    nch = bpw // ch
    idx3 = idx.reshape(nw, nch, ch)
    mesh = plsc.VectorSubcoreMesh(core_axis_name="c", subcore_axis_name="s")

    @functools.partial(
        pl.kernel,
        mesh=mesh,
        out_type=jax.ShapeDtypeStruct((B, D), jnp.float32),
        scratch_types=[
            pltpu.VMEM_SHARED((table.shape[0], D), jnp.float32),
            pltpu.VMEM((ch,), jnp.int32),
            pltpu.VMEM((ch,), jnp.int32),
            pltpu.VMEM((ch, D), jnp.float32),
            pltpu.VMEM((ch, D), jnp.float32),
            pltpu.SemaphoreType.DMA,
            pltpu.SemaphoreType.DMA,
            pltpu.SemaphoreType.DMA,
            pltpu.SemaphoreType.DMA,
            pltpu.SemaphoreType.DMA,
            pltpu.SemaphoreType.DMA,
        ],
    )
    def k(table_hbm, idx_hbm, out_hbm, table_sh, i0, i1, r0, r1,
          gi0, gi1, g0, g1, o0, o1):
        wid = lax.axis_index("s") * nc + lax.axis_index("c")
        base = wid * bpw
        idxs = (i0, i1)
        rows = (r0, r1)
        gi = (gi0, gi1)
        g = (g0, g1)
        o = (o0, o1)

        @pl.when(lax.axis_index("s") == 0)
        def _():
            pltpu.sync_copy(table_hbm, table_sh)
        plsc.subcore_barrier()

        def start_idx(j, b):
            pltpu.async_copy(idx_hbm.at[wid, j], idxs[b], gi[b])

        def wait_idx(b):
            pltpu.make_async_copy(idx_hbm.at[wid, 0], idxs[b], gi[b]).wait()

        def start_gather(b):
            pltpu.async_copy(table_sh.at[idxs[b]], rows[b], g[b])

        def wait_gather(b):
            pltpu.make_async_copy(table_hbm.at[i0], rows[b], g[b]).wait()

        def start_out(j, b):
            pltpu.async_copy(rows[b], out_hbm.at[pl.ds(base + j * ch, ch)],
                             o[b])

        def wait_out(b):
            pltpu.make_async_copy(table_hbm.at[i0], rows[b], o[b]).wait()

        start_idx(0, 0)
        start_idx(1, 1)

        def chunk(j, b):
            wait_idx(b)

            @pl.when(j >= 2)
            def _():
                wait_out(b)

            start_gather(b)
            wait_gather(b)
            start_out(j, b)

            @pl.when(j + 2 < nch)
            def _():
                start_idx(j + 2, b)

        def body(i, _):
            for b in (0, 1):
                chunk(i * 2 + b, b)
            return 0

        lax.fori_loop(0, nch // 2, body, 0, unroll=False)
        pltpu.make_async_copy(table_hbm.at[i0], rows[0], o[0]).wait()
        pltpu.make_async_copy(table_hbm.at[i0], rows[1], o[1]).wait()

    return k(table, idx3)


def _backbone_body(x_ref, w_ref, b_ref, fcw_ref, fcb_ref, pw_ref, pb_ref,
                   iv_ref, acc_ref):
    c = pl.program_id(0)

    @pl.when(c == 0)
    def _():
        acc_ref[...] = jnp.zeros_like(acc_ref)

    h = jnp.maximum(jnp.dot(x_ref[...], w_ref[...]) + b_ref[...], 0.0)
    acc_ref[...] += jnp.dot(jnp.ones((8, x_ref.shape[0]), jnp.float32), h)

    @pl.when(c == pl.num_programs(0) - 1)
    def _():
        m = acc_ref[...] * (1.0 / 3136.0)
        hh = jnp.maximum(jnp.dot(m, fcw_ref[...]) + fcb_ref[...], 0.0)
        iv_ref[...] = jnp.dot(hh, pw_ref[...]) + pb_ref[...]


def _node_enc_body(mp_ref, w1_ref, b1_ref, w2_ref, b2_ref, iv_ref,
                   ws_ref, wr_ref, x_ref, xs_ref, xr_ref):
    t = jnp.maximum(jnp.dot(mp_ref[...], w1_ref[...]) + b1_ref[...], 0.0)
    y = jnp.dot(t, w2_ref[...]) + b2_ref[...]
    x = _ln_rows(y) + iv_ref[0:1, :]
    x_ref[...] = x
    xs_ref[...] = jnp.dot(x, ws_ref[...])
    xr_ref[...] = jnp.dot(x, wr_ref[...])


def _edge_enc_body(wstart_ref, ms_ref, r_ref, mp_ref, w1_ref, b1_ref,
                   w2_ref, b2_ref, e0_ref):
    c = pl.program_id(0)
    ws = wstart_ref[c]
    r_loc = r_ref[...] - ws
    oh = (r_loc == lax.broadcasted_iota(jnp.int32, (EB, W), 1))
    mr = jnp.dot(oh.astype(jnp.float32), mp_ref[pl.ds(ws, W), :])
    d = ms_ref[:, 0:16] - mr
    nrm = jnp.sqrt(jnp.sum(d * d, axis=-1, keepdims=True))
    col3 = (lax.broadcasted_iota(jnp.int32, (EB, 16), 1) == 3)
    ef = d + jnp.where(col3, nrm, 0.0)
    t = jnp.maximum(jnp.dot(ef, w1_ref[...]) + b1_ref[...], 0.0)
    y = jnp.dot(t, w2_ref[...]) + b2_ref[...]
    e0_ref[...] = _ln_rows(y)


def _edge_step_body(wstart_ref, e_ref, gs_ref, r_ref, m_ref, xr_ref,
                    we_ref, w2_ref, b1_ref, b2_ref, aw_ref,
                    enew_ref, num_ref, den_ref):
    c = pl.program_id(0)

    @pl.when(c == 0)
    def _():
        num_ref[...] = jnp.zeros_like(num_ref)
        den_ref[...] = jnp.zeros_like(den_ref)

    ws = wstart_ref[c]
    e = e_ref[...]
    r_loc = r_ref[...] - ws
    mask = m_ref[...]
    oh = (r_loc == lax.broadcasted_iota(jnp.int32, (EB, W), 1))
    oh = oh.astype(jnp.float32)
    xr_win = xr_ref[pl.ds(ws, W), :]
    gr = jnp.dot(oh, xr_win)
    t = jnp.maximum(jnp.dot(e, we_ref[...]) + gs_ref[...] + gr + b1_ref[...],
                    0.0)
    y = jnp.dot(t, w2_ref[...]) + b2_ref[...]
    e_new = _ln_rows(y) + e
    enew_ref[...] = e_new
    l = jnp.sum(e_new * aw_ref[...], axis=-1, keepdims=True)
    l = jnp.where(l >= 0, l, 0.2 * l)
    u = jnp.where(mask > 0.5, jnp.exp(l), 0.0)
    w = e_new * u
    dn = (((0,), (0,)), ((), ()))
    pnum = lax.dot_general(oh, w, dn)
    pden = lax.dot_general(oh, jnp.broadcast_to(u, (EB, 8)), dn)
    num_ref[pl.ds(ws, W), :] += pnum
    den_ref[pl.ds(ws, W), :] += pden


def _node_step_body(num_ref, den_ref, x_ref, wx_ref, wa_ref, b1_ref,
                    w2_ref, b2_ref, ws_ref, wr_ref,
                    xn_ref, xs_ref, xr_ref):
    agg = num_ref[...] / jnp.maximum(den_ref[:, 0:1], 1e-20)
    x = x_ref[...]
    t = jnp.maximum(jnp.dot(x, wx_ref[...]) + jnp.dot(agg, wa_ref[...])
                    + b1_ref[...], 0.0)
    y = jnp.dot(t, w2_ref[...]) + b2_ref[...]
    x_new = _ln_rows(y) + x
    xn_ref[...] = x_new
    xs_ref[...] = jnp.dot(x_new, ws_ref[...])
    xr_ref[...] = jnp.dot(x_new, wr_ref[...])


def _dec_body(x_ref, w1_ref, b1_ref, w2_ref, b2_ref, o_ref):
    t = jnp.maximum(jnp.dot(x_ref[...], w1_ref[...]) + b1_ref[...], 0.0)
    o_ref[...] = jnp.dot(t, w2_ref[...]) + b2_ref[...]


def _const2(shape):
    return pl.BlockSpec(shape, lambda *_: (0, 0))


def _rowblk(bs, nl):
    return pl.BlockSpec((bs, nl), lambda c, *_: (c, 0))


def kernel(batch, mesh_pos, edge_idx, params):
    p = params
    f32 = jnp.float32

    senders = jnp.concatenate([edge_idx[:, 0], edge_idx[:, 1]], axis=0)
    receivers = jnp.concatenate([edge_idx[:, 1], edge_idx[:, 0]], axis=0)
    order = jnp.argsort(receivers)
    r_s = receivers[order].astype(jnp.int32)
    s_s = senders[order].astype(jnp.int32)
    deg = jnp.zeros((N,), jnp.int32).at[receivers].add(1)
    start = jnp.cumsum(deg) - deg
    pd = jnp.maximum(8, ((deg + 7) // 8) * 8)
    cum = jnp.cumsum(pd)
    slot0 = cum - pd
    pos = jnp.arange(E2, dtype=jnp.int32) + (slot0 - start)[r_s]
    r_pad = jax.lax.cummax(
        jnp.zeros((EPAD,), jnp.int32).at[slot0].set(
            jnp.arange(N, dtype=jnp.int32)))
    sm = jnp.zeros((EPAD,), jnp.int32).at[pos].set(s_s + 1)
    s_pad = jnp.maximum(sm - 1, 0)
    maskf = (sm > 0).astype(f32)
    wstart = ((r_pad[::EB] // 8) * 8).astype(jnp.int32)
    r_col = r_pad.reshape(EPAD, 1)
    m_col = maskf.reshape(EPAD, 1)

    we = p['mp_e_w1'][0:L]
    wsnd = p['mp_e_w1'][L:2 * L]
    wrcv = p['mp_e_w1'][2 * L:3 * L]
    wnx = p['mp_n_w1'][0:L]
    wna = p['mp_n_w1'][L:2 * L]
    b1e = p['mp_e_b1'].reshape(1, L)
    b2e = p['mp_e_b2'].reshape(1, L)
    b1n = p['mp_n_b1'].reshape(1, L)
    b2n = p['mp_n_b2'].reshape(1, L)
    aw = p['att_w'].reshape(1, L)
    enw1 = jnp.zeros((16, L), f32).at[0:3].set(p['enc_n_w1'])
    enb1 = p['enc_n_b1'].reshape(1, L)
    enw2 = p['enc_n_w2']
    enb2 = p['enc_n_b2'].reshape(1, L)
    eew1 = jnp.zeros((16, L), f32).at[0:4].set(p['enc_e_w1'])
    eeb1 = p['enc_e_b1'].reshape(1, L)
    eew2 = p['enc_e_w2']
    eeb2 = p['enc_e_b2'].reshape(1, L)
    dw1 = p['dec_w1']
    db1 = p['dec_b1'].reshape(1, L)
    dw2 = jnp.zeros((L, L), f32).at[:, 0:3].set(p['dec_w2'])
    db2 = jnp.zeros((1, L), f32).at[0, 0:3].set(p['dec_b2'])

    xpad = jnp.pad(batch, ((0, 0), (0, 0), (1, 2), (1, 2)))
    cols = []
    for ky in range(7):
        for kx in range(7):
            cols.append(xpad[0, :, ky:ky + 221:4, kx:kx + 221:4]
                        .reshape(3, 3136))
    X = jnp.concatenate(cols, axis=0)
    Wc = jnp.transpose(p['conv_w'], (2, 3, 1, 0)).reshape(147, 64)
    Xp = jnp.zeros((3136, 160), f32).at[:, 0:147].set(X.T)
    Wp = jnp.zeros((160, 64), f32).at[0:147].set(Wc)
    cb = p['conv_b'].reshape(1, 64)
    fcw = p['fc_w']
    fcb = p['fc_b'].reshape(1, 512)
    pw = p['img_proj_w']
    pb = p['img_proj_b'].reshape(1, L)

    iv = pl.pallas_call(
        _backbone_body,
        grid=(7,),
        in_specs=[
            pl.BlockSpec((448, 160), lambda c: (c, 0)),
            _const2((160, 64)), _const2((1, 64)),
            _const2((64, 512)), _const2((1, 512)),
            _const2((512, L)), _const2((1, L)),
        ],
        out_specs=_const2((8, L)),
        out_shape=jax.ShapeDtypeStruct((8, L), f32),
        scratch_shapes=[pltpu.VMEM((8, 64), f32)],
    )(Xp, Wp, cb, fcw, fcb, pw, pb)

    mp16 = jnp.zeros((NPAD, 16), f32).at[0:N, 0:3].set(mesh_pos)
    x0, xs0, xr0 = pl.pallas_call(
        _node_enc_body,
        grid=(NNBLK,),
        in_specs=[
            _rowblk(NB, 16),
            _const2((16, L)), _const2((1, L)), _const2((L, L)), _const2((1, L)),
            _const2((8, L)), _const2((L, L)), _const2((L, L)),
        ],
        out_specs=[_rowblk(NB, L)] * 3,
        out_shape=[jax.ShapeDtypeStruct((NPAD, L), f32)] * 3,
    )(mp16, enw1, enb1, enw2, enb2, iv, wsnd, wrcv)

    mp128 = jnp.zeros((NPAD, L), f32).at[0:N, 0:3].set(mesh_pos)
    mp_s = _sc_gather(mp128, s_pad)
    e0 = pl.pallas_call(
        _edge_enc_body,
        grid_spec=pltpu.PrefetchScalarGridSpec(
            num_scalar_prefetch=1,
            grid=(NBLK,),
            in_specs=[
                _rowblk(EB, L), _rowblk(EB, 1),
                _const2((NPAD, 16)),
                _const2((16, L)), _const2((1, L)), _const2((L, L)),
                _const2((1, L)),
            ],
            out_specs=_rowblk(EB, L),
        ),
        out_shape=jax.ShapeDtypeStruct((EPAD, L), f32),
    )(wstart, mp_s, r_col, mp16, eew1, eeb1, eew2, eeb2)

    edge_call = pl.pallas_call(
        _edge_step_body,
        grid_spec=pltpu.PrefetchScalarGridSpec(
            num_scalar_prefetch=1,
            grid=(NBLK,),
            in_specs=[
                _rowblk(EB, L), _rowblk(EB, L),
                _rowblk(EB, 1), _rowblk(EB, 1),
                _const2((NPAD, L)),
                _const2((L, L)), _const2((L, L)),
                _const2((1, L)), _const2((1, L)), _const2((1, L)),
            ],
            out_specs=[
                _rowblk(EB, L),
                _const2((NPAD, L)),
                _const2((NPAD, 8)),
            ],
        ),
        out_shape=[
            jax.ShapeDtypeStruct((EPAD, L), f32),
            jax.ShapeDtypeStruct((NPAD, L), f32),
            jax.ShapeDtypeStruct((NPAD, 8), f32),
        ],
        input_output_aliases={1: 0},
    )

    node_call = pl.pallas_call(
        _node_step_body,
        grid=(NNBLK,),
        in_specs=[
            _rowblk(NB, L), _rowblk(NB, 8), _rowblk(NB, L),
            _const2((L, L)), _const2((L, L)), _const2((1, L)),
            _const2((L, L)), _const2((1, L)),
            _const2((L, L)), _const2((L, L)),
        ],
        out_specs=[_rowblk(NB, L)] * 3,
        out_shape=[jax.ShapeDtypeStruct((NPAD, L), f32)] * 3,
        input_output_aliases={2: 0},
    )

    def step(carry, _):
        x, e, xs, xr = carry
        gs = _sc_gather(xs, s_pad)
        e_new, num, den = edge_call(wstart, e, gs, r_col, m_col, xr,
                                    we, p['mp_e_w2'], b1e, b2e, aw)
        x_new, xs2, xr2 = node_call(num, den, x, wnx, wna, b1n,
                                    p['mp_n_w2'], b2n, wsnd, wrcv)
        return (x_new, e_new, xs2, xr2), None

    (x, e, xs, xr), _ = lax.scan(step, (x0, e0, xs0, xr0), None, length=STEPS)

    pred = pl.pallas_call(
        _dec_body,
        grid=(NNBLK,),
        in_specs=[
            _rowblk(NB, L),
            _const2((L, L)), _const2((1, L)), _const2((L, L)), _const2((1, L)),
        ],
        out_specs=_rowblk(NB, L),
        out_shape=jax.ShapeDtypeStruct((NPAD, L), f32),
    )(x, dw1, db1, dw2, db2)

    return pred[0:N, 0:3]
